# baseline (speedup 1.0000x reference)
import functools

import jax
import jax.numpy as jnp
import numpy as np
from jax.experimental import pallas as pl
from jax.experimental.pallas import tpu as pltpu
from jax.experimental.shard_map import shard_map
from jax.sharding import Mesh, PartitionSpec as P

B, D, O, E = 16384, 768, 768, 8
BLOCK_B = 1024


def _body(x_ref, ws_ref, bs_ref, w_ref, bp_ref, o_ref, wb_ref):
    @pl.when(pl.program_id(0) == 0)
    def _cast_weights():
        wb_ref[...] = w_ref[...].astype(jnp.bfloat16)

    x = x_ref[...].astype(jnp.bfloat16)
    logits = jnp.dot(
        x, ws_ref[...].astype(jnp.bfloat16), preferred_element_type=jnp.float32
    )
    logits = logits + bs_ref[...]
    p = jax.nn.softmax(logits, axis=-1)
    acc = jnp.zeros((x.shape[0], O), jnp.float32)
    for e in range(E):
        y_e = jnp.dot(x, wb_ref[e], preferred_element_type=jnp.float32)
        acc += p[:, e : e + 1] * (y_e + bp_ref[e : e + 1, :])
    o_ref[...] = acc


def _drn_block(x, W_sel, b_sel, W_pops, b_pops):
    grid = (x.shape[0] // BLOCK_B,)
    return pl.pallas_call(
        _body,
        grid=grid,
        in_specs=[
            pl.BlockSpec((BLOCK_B, D), lambda i: (i, 0)),
            pl.BlockSpec((D, E), lambda i: (0, 0)),
            pl.BlockSpec((1, E), lambda i: (0, 0)),
            pl.BlockSpec((E, D, O), lambda i: (0, 0, 0)),
            pl.BlockSpec((E, O), lambda i: (0, 0)),
        ],
        out_specs=pl.BlockSpec((BLOCK_B, O), lambda i: (i, 0)),
        out_shape=jax.ShapeDtypeStruct((x.shape[0], O), jnp.float32),
        scratch_shapes=[pltpu.VMEM((E, D, O), jnp.bfloat16)],
        compiler_params=pltpu.CompilerParams(
            dimension_semantics=("arbitrary",),
        ),
    )(x, W_sel, b_sel, W_pops, b_pops)


@functools.lru_cache(maxsize=1)
def _sharded_fn():
    devs = jax.devices()
    n = 2 if len(devs) >= 2 else 1
    mesh = Mesh(np.array(devs[:n]), ("dp",))
    return shard_map(
        _drn_block,
        mesh=mesh,
        in_specs=(P("dp", None), P(None, None), P(None, None),
                  P(None, None, None), P(None, None)),
        out_specs=P("dp", None),
        check_rep=False,
    )


def kernel(x, W_sel, b_sel, W_pops, b_pops):
    return _sharded_fn()(x, W_sel, b_sel.reshape(1, E), W_pops, b_pops)


# speedup vs baseline: 2.6604x; 2.6604x over previous
import jax
import jax.numpy as jnp
from jax.experimental import pallas as pl
from jax.experimental.pallas import tpu as pltpu

B, D, O, E = 16384, 768, 768, 8
BLOCK_B = 2048
NSPLIT = 3
NCHUNK = O // NSPLIT


def _body(x_ref, ws_ref, bs_ref, w_ref, bp_ref, o_ref, wb_ref, p_ref):
    n = pl.program_id(0)
    i = pl.program_id(1)

    @pl.when(i == 0)
    def _cast_weights():
        wb_ref[...] = w_ref[...].astype(jnp.bfloat16)

    x = x_ref[...].astype(jnp.bfloat16)

    @pl.when(n == 0)
    def _probs():
        logits = jnp.dot(
            x, ws_ref[...].astype(jnp.bfloat16), preferred_element_type=jnp.float32
        )
        logits = logits + bs_ref[...]
        p_ref[pl.ds(i * BLOCK_B, BLOCK_B), :] = jax.nn.softmax(
            logits, axis=-1
        ).astype(jnp.bfloat16)

    p = p_ref[pl.ds(i * BLOCK_B, BLOCK_B), :].astype(jnp.float32)
    acc = jnp.zeros((BLOCK_B, NCHUNK), jnp.float32)
    for e in range(E):
        y_e = jnp.dot(x, wb_ref[e], preferred_element_type=jnp.float32)
        acc += p[:, e : e + 1] * (y_e + bp_ref[e : e + 1, :])
    o_ref[...] = acc


def kernel(x, W_sel, b_sel, W_pops, b_pops):
    grid = (NSPLIT, B // BLOCK_B)
    return pl.pallas_call(
        _body,
        grid=grid,
        in_specs=[
            pl.BlockSpec((BLOCK_B, D), lambda n, i: (i, 0)),
            pl.BlockSpec((D, E), lambda n, i: (0, 0)),
            pl.BlockSpec((1, E), lambda n, i: (0, 0)),
            pl.BlockSpec((E, D, NCHUNK), lambda n, i: (0, 0, n)),
            pl.BlockSpec((E, NCHUNK), lambda n, i: (0, n)),
        ],
        out_specs=pl.BlockSpec((BLOCK_B, NCHUNK), lambda n, i: (i, n)),
        out_shape=jax.ShapeDtypeStruct((B, O), jnp.float32),
        scratch_shapes=[
            pltpu.VMEM((E, D, NCHUNK), jnp.bfloat16),
            pltpu.VMEM((B, E), jnp.bfloat16),
        ],
        compiler_params=pltpu.CompilerParams(
            dimension_semantics=("arbitrary", "arbitrary"),
        ),
    )(x, W_sel, b_sel.reshape(1, E), W_pops, b_pops)


# speedup vs baseline: 2.8211x; 1.0604x over previous
import jax
import jax.numpy as jnp
from jax.experimental import pallas as pl
from jax.experimental.pallas import tpu as pltpu

B, D, O, E = 16384, 768, 768, 8
BLOCK_B = 1024


def _body(x_ref, ws_ref, bs_ref, w_ref, bp_ref, o_ref):
    x = x_ref[...]
    logits = jnp.dot(x, ws_ref[...], preferred_element_type=jnp.float32)
    logits = logits + bs_ref[...]
    p = jax.nn.softmax(logits, axis=-1)
    acc = jnp.zeros((x.shape[0], O), jnp.float32)
    for e in range(E):
        y_e = jnp.dot(x, w_ref[e], preferred_element_type=jnp.float32)
        acc += p[:, e : e + 1] * (y_e + bp_ref[e : e + 1, :])
    o_ref[...] = acc


def kernel(x, W_sel, b_sel, W_pops, b_pops):
    grid = (B // BLOCK_B,)
    return pl.pallas_call(
        _body,
        grid=grid,
        in_specs=[
            pl.BlockSpec((BLOCK_B, D), lambda i: (i, 0)),
            pl.BlockSpec((D, E), lambda i: (0, 0)),
            pl.BlockSpec((1, E), lambda i: (0, 0)),
            pl.BlockSpec((E, D, O), lambda i: (0, 0, 0)),
            pl.BlockSpec((E, O), lambda i: (0, 0)),
        ],
        out_specs=pl.BlockSpec((BLOCK_B, O), lambda i: (i, 0)),
        out_shape=jax.ShapeDtypeStruct((B, O), jnp.float32),
        compiler_params=pltpu.CompilerParams(
            dimension_semantics=("arbitrary",),
        ),
    )(x, W_sel, b_sel.reshape(1, E), W_pops, b_pops)


# speedup vs baseline: 2.8345x; 1.0047x over previous
import jax
import jax.numpy as jnp
from jax.experimental import pallas as pl
from jax.experimental.pallas import tpu as pltpu

B, D, O, E = 16384, 768, 768, 8
BLOCK_B = 1024


def _body(x_ref, ws_ref, bs_ref, w_ref, bp_ref, o_ref, wb_ref):
    @pl.when(pl.program_id(0) == 0)
    def _cast_weights():
        wb_ref[...] = w_ref[...].astype(jnp.bfloat16)

    x = x_ref[...].astype(jnp.bfloat16)
    logits = jnp.dot(
        x, ws_ref[...].astype(jnp.bfloat16), preferred_element_type=jnp.float32
    )
    logits = logits + bs_ref[...]
    p = jax.nn.softmax(logits, axis=-1)
    acc = jnp.zeros((x.shape[0], O), jnp.float32)
    for e in range(E):
        y_e = jnp.dot(x, wb_ref[e], preferred_element_type=jnp.float32)
        acc += p[:, e : e + 1] * (y_e + bp_ref[e : e + 1, :])
    o_ref[...] = acc


def kernel(x, W_sel, b_sel, W_pops, b_pops):
    grid = (B // BLOCK_B,)
    return pl.pallas_call(
        _body,
        grid=grid,
        in_specs=[
            pl.BlockSpec((BLOCK_B, D), lambda i: (i, 0)),
            pl.BlockSpec((D, E), lambda i: (0, 0)),
            pl.BlockSpec((1, E), lambda i: (0, 0)),
            pl.BlockSpec((E, D, O), lambda i: (0, 0, 0)),
            pl.BlockSpec((E, O), lambda i: (0, 0)),
        ],
        out_specs=pl.BlockSpec((BLOCK_B, O), lambda i: (i, 0)),
        out_shape=jax.ShapeDtypeStruct((B, O), jnp.float32),
        scratch_shapes=[pltpu.VMEM((E, D, O), jnp.bfloat16)],
        compiler_params=pltpu.CompilerParams(
            dimension_semantics=("arbitrary",),
        ),
    )(x, W_sel, b_sel.reshape(1, E), W_pops, b_pops)
